# baseline (speedup 1.0000x reference)
import functools

import jax
import jax.numpy as jnp
from jax import lax
from jax.experimental import pallas as pl
from jax.experimental.pallas import tpu as pltpu
from jax.experimental.pallas import tpu_sc as plsc

N = 10000
D = 128
H = 32
C = 16
FH = 64

NC = 2
NS = 16
NW = NC * NS
CHUNK = 128
LANES = 16
WPAD = 128

ROWS_PER_TILE = 632
TAIL_ROWS = N - 15 * ROWS_PER_TILE
GRP = 4


def _make_spmm(width: int, n_chunks: int):
  mesh = plsc.VectorSubcoreMesh(core_axis_name="c", subcore_axis_name="s")

  @functools.partial(
      pl.kernel,
      out_type=jax.ShapeDtypeStruct((NC, N, WPAD), jnp.float32),
      mesh=mesh,
      scratch_types=[
          pltpu.VMEM_SHARED((N, WPAD), jnp.float32),
          pltpu.VMEM((2 * GRP, CHUNK), jnp.int32),
          pltpu.VMEM((CHUNK, LANES), jnp.float32),
          pltpu.VMEM((CHUNK, WPAD), jnp.float32),
          pltpu.VMEM((CHUNK, WPAD), jnp.float32),
          pltpu.SemaphoreType.DMA,
          pltpu.SemaphoreType.DMA,
          pltpu.SemaphoreType.DMA,
      ],
  )
  def spmm(sup_hbm, idx_hbm, wexp_hbm, out_hbm,
           accum, idx_v, wexp_v, rows_v0, rows_v1,
           sem_r0, sem_r1, sem_w):
    cid = lax.axis_index("c")
    sid = lax.axis_index("s")
    wid = cid * NS + sid
    row0 = sid * ROWS_PER_TILE

    zero16 = jnp.zeros((LANES,), jnp.float32)

    def zrow(i, carry):
      for jj in range(WPAD // LANES):
        rows_v0[i, pl.ds(jj * LANES, LANES)] = zero16
      return carry

    lax.fori_loop(0, CHUNK, zrow, 0)

    def _acc_rows(fn):
      @pl.when(sid < NS - 1)
      def _():
        for k in range(5):
          nrows = min(CHUNK, ROWS_PER_TILE - k * CHUNK)
          fn(row0 + k * CHUNK, nrows)

      @pl.when(sid == NS - 1)
      def _():
        for k in range(5):
          nrows = min(CHUNK, TAIL_ROWS - k * CHUNK)
          if nrows <= 0:
            break
          fn(row0 + k * CHUNK, nrows)

    _acc_rows(lambda r, nrows: pltpu.sync_copy(
        rows_v0.at[pl.ds(0, nrows)], accum.at[pl.ds(r, nrows)]))
    plsc.subcore_barrier()

    ebase = wid * n_chunks

    def mult(rows_v, wexp_v):
      def mbody(eb, carry):
        for j in range(8):
          e = eb * 8 + j
          ws = wexp_v[e, :]
          for f in range(width // LANES):
            sl = pl.ds(f * LANES, LANES)
            rows_v[e, sl] = rows_v[e, sl] * ws
        return carry

      lax.fori_loop(0, CHUNK // 8, mbody, 0)

    def group_body(g, carry):
      gbase = ebase + g * GRP
      pltpu.sync_copy(idx_hbm.at[pl.ds(2 * gbase, 2 * GRP)], idx_v)

      def pair_body(p, carry2):
        c0 = 2 * p
        c1 = c0 + 1
        d0 = pltpu.async_copy(sup_hbm.at[idx_v.at[2 * c0]], rows_v0, sem_r0)
        d1 = pltpu.async_copy(sup_hbm.at[idx_v.at[2 * c1]], rows_v1, sem_r1)
        e0 = pltpu.async_copy(
            wexp_hbm.at[pl.ds((gbase + c0) * CHUNK, CHUNK)], wexp_v, sem_w)
        d0.wait()
        e0.wait()
        mult(rows_v0, wexp_v)
        pltpu.sync_copy(rows_v0, accum.at[idx_v.at[2 * c0 + 1]], add=True)
        e1 = pltpu.async_copy(
            wexp_hbm.at[pl.ds((gbase + c1) * CHUNK, CHUNK)], wexp_v, sem_w)
        d1.wait()
        e1.wait()
        mult(rows_v1, wexp_v)
        pltpu.sync_copy(rows_v1, accum.at[idx_v.at[2 * c1 + 1]], add=True)
        return carry2

      lax.fori_loop(0, GRP // 2, pair_body, 0)
      return carry

    lax.fori_loop(0, n_chunks // GRP, group_body, 0)
    plsc.subcore_barrier()

    def _copy_out(r, nrows):
      pltpu.sync_copy(accum.at[pl.ds(r, nrows)], rows_v0.at[pl.ds(0, nrows)])
      pltpu.sync_copy(rows_v0.at[pl.ds(0, nrows)],
                      out_hbm.at[cid, pl.ds(r, nrows)])

    _acc_rows(_copy_out)

  return spmm


def _chunks_per_worker(E: int) -> int:
  return -(-(-(-E // (NW * CHUNK))) // 8) * 8


_N_CHUNKS = _chunks_per_worker(320000)
_spmm64 = _make_spmm(2 * H, _N_CHUNKS)
_spmm32 = _make_spmm(2 * C, _N_CHUNKS)


def _tc_support1(x0, x1, W1_0, W1_1):
  def body(x0_ref, x1_ref, w0_ref, w1_ref, o_ref):
    a = jnp.dot(x0_ref[...], w0_ref[...], preferred_element_type=jnp.float32)
    b = jnp.dot(x1_ref[...], w1_ref[...], preferred_element_type=jnp.float32)
    z = jnp.zeros((a.shape[0], WPAD - 2 * H), jnp.float32)
    o_ref[...] = jnp.concatenate([a, b, z], axis=1)

  return pl.pallas_call(
      body,
      out_shape=jax.ShapeDtypeStruct((N, WPAD), jnp.float32),
  )(x0, x1, W1_0, W1_1)


def _tc_wexp(w2col):
  BLK = 4096
  Ep = w2col.shape[0]

  def body(w_ref, o_ref):
    o_ref[...] = jnp.broadcast_to(w_ref[...], (BLK, LANES))

  return pl.pallas_call(
      body,
      grid=(Ep // BLK,),
      in_specs=[pl.BlockSpec((BLK, 1), lambda i: (i, 0))],
      out_specs=pl.BlockSpec((BLK, LANES), lambda i: (i, 0)),
      out_shape=jax.ShapeDtypeStruct((Ep, LANES), jnp.float32),
  )(w2col)


def _tc_support2(p, b1_0, b1_1, W2_0, W2_1):
  def body(p_ref, b0_ref, b1_ref, w0_ref, w1_ref, o_ref):
    s = p_ref[0, :N] + p_ref[1, :N]
    h0 = jnp.maximum(s[:, :H] + b0_ref[...], 0.0)
    h1 = jnp.maximum(s[:, H:2 * H] + b1_ref[...], 0.0)
    a = jnp.dot(h0, w0_ref[...], preferred_element_type=jnp.float32)
    b = jnp.dot(h1, w1_ref[...], preferred_element_type=jnp.float32)
    z = jnp.zeros((a.shape[0], WPAD - 2 * C), jnp.float32)
    o_ref[...] = jnp.concatenate([a, b, z], axis=1)

  return pl.pallas_call(
      body,
      out_shape=jax.ShapeDtypeStruct((N, WPAD), jnp.float32),
  )(p, b1_0, b1_1, W2_0, W2_1)


def _tc_fuse(p2, b2_0, b2_1, A1, a1b, A2, A3r):
  def body(p_ref, b0_ref, b1_ref, A1_ref, a1b_ref, A2_ref, A3_ref, o_ref):
    s = p_ref[0, :N] + p_ref[1, :N]
    o0 = s[:, :C] + b0_ref[...]
    o1 = s[:, C:2 * C] + b1_ref[...]

    def logsm(o):
      m = jnp.max(o, axis=1, keepdims=True)
      return o - m - jnp.log(jnp.sum(jnp.exp(o - m), axis=1, keepdims=True))

    z0 = logsm(o0)
    z1 = logsm(o1)

    def att(z):
      h = jnp.tanh(
          jnp.dot(z, A1_ref[...], preferred_element_type=jnp.float32)
          + a1b_ref[...])
      g = jnp.tanh(jnp.dot(h, A2_ref[...], preferred_element_type=jnp.float32))
      return jnp.sum(g * A3_ref[...], axis=1, keepdims=True)

    w0 = att(z0)
    w1 = att(z1)
    m = jnp.maximum(w0, w1)
    e0 = jnp.exp(w0 - m)
    e1 = jnp.exp(w1 - m)
    o_ref[...] = (e0 * z0 + e1 * z1) / (e0 + e1)

  return pl.pallas_call(
      body,
      out_shape=jax.ShapeDtypeStruct((N, C), jnp.float32),
  )(p2, b2_0, b2_1, A1, a1b, A2, A3r)


def kernel(x0, x1, edge_index, edge_weight, W1_0, b1_0, W2_0, b2_0,
           W1_1, b1_1, W2_1, b2_1, A1, a1b, A2, A3):
  src = edge_index[1].astype(jnp.int32)
  dst = edge_index[0].astype(jnp.int32)
  w = edge_weight.astype(jnp.float32)
  E = src.shape[0]
  n_chunks = _chunks_per_worker(E)
  pad = NW * n_chunks * CHUNK - E
  src2 = jnp.pad(src, (0, pad)).reshape(NW * n_chunks, CHUNK)
  dst2 = jnp.pad(dst, (0, pad)).reshape(NW * n_chunks, CHUNK)
  idx2 = jnp.stack([src2, dst2], axis=1).reshape(2 * NW * n_chunks, CHUNK)
  w2col = jnp.pad(w, (0, pad)).reshape(-1, 1)

  wexp = _tc_wexp(w2col)
  sup1 = _tc_support1(x0, x1, W1_0, W1_1)
  p1 = _spmm64(sup1, idx2, wexp)
  sup2 = _tc_support2(p1, b1_0.reshape(1, H), b1_1.reshape(1, H), W2_0, W2_1)
  p2 = _spmm32(sup2, idx2, wexp)
  return _tc_fuse(p2, b2_0.reshape(1, C), b2_1.reshape(1, C),
                  A1, a1b.reshape(1, FH), A2, A3.reshape(1, 2 * C))


# speedup vs baseline: 1.0320x; 1.0320x over previous
import functools

import jax
import jax.numpy as jnp
from jax import lax
from jax.experimental import pallas as pl
from jax.experimental.pallas import tpu as pltpu
from jax.experimental.pallas import tpu_sc as plsc

N = 10000
D = 128
H = 32
C = 16
FH = 64

NC = 2
NS = 16
NW = NC * NS
CHUNK = 128
LANES = 16
WPAD = 128

ROWS_PER_TILE = 632
TAIL_ROWS = N - 15 * ROWS_PER_TILE
GRP = 4


def _make_spmm(width: int, n_chunks: int):
  mesh = plsc.VectorSubcoreMesh(core_axis_name="c", subcore_axis_name="s")

  @functools.partial(
      pl.kernel,
      out_type=jax.ShapeDtypeStruct((NC, N, WPAD), jnp.float32),
      mesh=mesh,
      scratch_types=[
          pltpu.VMEM_SHARED((N, WPAD), jnp.float32),
          pltpu.VMEM((2 * GRP, CHUNK), jnp.int32),
          pltpu.VMEM((CHUNK // 2, LANES), jnp.float32),
          pltpu.VMEM((CHUNK // 2, LANES), jnp.float32),
          pltpu.VMEM((CHUNK, WPAD), jnp.float32),
          pltpu.VMEM((CHUNK, WPAD), jnp.float32),
          pltpu.SemaphoreType.DMA,
          pltpu.SemaphoreType.DMA,
          pltpu.SemaphoreType.DMA,
          pltpu.SemaphoreType.DMA,
      ],
  )
  def spmm(sup_hbm, idx_hbm, wexp_hbm, out_hbm,
           accum, idx_v, wexp_va, wexp_vb, rows_v0, rows_v1,
           sem_r0, sem_r1, sem_wa, sem_wb):
    cid = lax.axis_index("c")
    sid = lax.axis_index("s")
    wid = cid * NS + sid
    row0 = sid * ROWS_PER_TILE

    zero16 = jnp.zeros((LANES,), jnp.float32)

    def zrow(i, carry):
      for jj in range(WPAD // LANES):
        rows_v0[i, pl.ds(jj * LANES, LANES)] = zero16
      return carry

    lax.fori_loop(0, CHUNK, zrow, 0)

    def _acc_rows(fn):
      @pl.when(sid < NS - 1)
      def _():
        for k in range(5):
          nrows = min(CHUNK, ROWS_PER_TILE - k * CHUNK)
          fn(row0 + k * CHUNK, nrows)

      @pl.when(sid == NS - 1)
      def _():
        for k in range(5):
          nrows = min(CHUNK, TAIL_ROWS - k * CHUNK)
          if nrows <= 0:
            break
          fn(row0 + k * CHUNK, nrows)

    _acc_rows(lambda r, nrows: pltpu.sync_copy(
        rows_v0.at[pl.ds(0, nrows)], accum.at[pl.ds(r, nrows)]))
    plsc.subcore_barrier()

    ebase = wid * n_chunks

    HALF = CHUNK // 2

    def mult_half(rows_v, wexp_v, base):
      def mbody(eb, carry):
        for j in range(16):
          e = eb * 16 + j
          ws = wexp_v[e, :]
          for f in range(width // LANES):
            sl = pl.ds(f * LANES, LANES)
            rows_v[base + e, sl] = rows_v[base + e, sl] * ws
        return carry

      lax.fori_loop(0, HALF // 16, mbody, 0)

    def group_body(g, carry):
      gbase = ebase + g * GRP
      pltpu.sync_copy(idx_hbm.at[pl.ds(2 * gbase, 2 * GRP)], idx_v)

      def fetch_w(c):
        ea = pltpu.async_copy(
            wexp_hbm.at[pl.ds((gbase + c) * CHUNK, HALF)], wexp_va, sem_wa)
        eb = pltpu.async_copy(
            wexp_hbm.at[pl.ds((gbase + c) * CHUNK + HALF, HALF)],
            wexp_vb, sem_wb)
        return ea, eb

      def pair_body(p, carry2):
        c0 = 2 * p
        c1 = c0 + 1
        d0 = pltpu.async_copy(sup_hbm.at[idx_v.at[2 * c0]], rows_v0, sem_r0)
        d1 = pltpu.async_copy(sup_hbm.at[idx_v.at[2 * c1]], rows_v1, sem_r1)
        ea, eb = fetch_w(c0)
        d0.wait()
        ea.wait()
        mult_half(rows_v0, wexp_va, 0)
        eb.wait()
        mult_half(rows_v0, wexp_vb, HALF)
        ea, eb = fetch_w(c1)
        pltpu.sync_copy(rows_v0, accum.at[idx_v.at[2 * c0 + 1]], add=True)
        d1.wait()
        ea.wait()
        mult_half(rows_v1, wexp_va, 0)
        eb.wait()
        mult_half(rows_v1, wexp_vb, HALF)
        pltpu.sync_copy(rows_v1, accum.at[idx_v.at[2 * c1 + 1]], add=True)
        return carry2

      lax.fori_loop(0, GRP // 2, pair_body, 0)
      return carry

    lax.fori_loop(0, n_chunks // GRP, group_body, 0)
    plsc.subcore_barrier()

    def _copy_out(r, nrows):
      pltpu.sync_copy(accum.at[pl.ds(r, nrows)], rows_v0.at[pl.ds(0, nrows)])
      pltpu.sync_copy(rows_v0.at[pl.ds(0, nrows)],
                      out_hbm.at[cid, pl.ds(r, nrows)])

    _acc_rows(_copy_out)

  return spmm


def _chunks_per_worker(E: int) -> int:
  return -(-(-(-E // (NW * CHUNK))) // 8) * 8


_N_CHUNKS = _chunks_per_worker(320000)
_spmm64 = _make_spmm(2 * H, _N_CHUNKS)
_spmm32 = _make_spmm(2 * C, _N_CHUNKS)


def _tc_support1(x0, x1, W1_0, W1_1):
  def body(x0_ref, x1_ref, w0_ref, w1_ref, o_ref):
    a = jnp.dot(x0_ref[...], w0_ref[...], preferred_element_type=jnp.float32)
    b = jnp.dot(x1_ref[...], w1_ref[...], preferred_element_type=jnp.float32)
    z = jnp.zeros((a.shape[0], WPAD - 2 * H), jnp.float32)
    o_ref[...] = jnp.concatenate([a, b, z], axis=1)

  return pl.pallas_call(
      body,
      out_shape=jax.ShapeDtypeStruct((N, WPAD), jnp.float32),
  )(x0, x1, W1_0, W1_1)


def _tc_wexp(w2col):
  BLK = 4096
  Ep = w2col.shape[0]

  def body(w_ref, o_ref):
    o_ref[...] = jnp.broadcast_to(w_ref[...], (BLK, LANES))

  return pl.pallas_call(
      body,
      grid=(Ep // BLK,),
      in_specs=[pl.BlockSpec((BLK, 1), lambda i: (i, 0))],
      out_specs=pl.BlockSpec((BLK, LANES), lambda i: (i, 0)),
      out_shape=jax.ShapeDtypeStruct((Ep, LANES), jnp.float32),
  )(w2col)


def _tc_support2(p, b1_0, b1_1, W2_0, W2_1):
  def body(p_ref, b0_ref, b1_ref, w0_ref, w1_ref, o_ref):
    s = p_ref[0, :N] + p_ref[1, :N]
    h0 = jnp.maximum(s[:, :H] + b0_ref[...], 0.0)
    h1 = jnp.maximum(s[:, H:2 * H] + b1_ref[...], 0.0)
    a = jnp.dot(h0, w0_ref[...], preferred_element_type=jnp.float32)
    b = jnp.dot(h1, w1_ref[...], preferred_element_type=jnp.float32)
    z = jnp.zeros((a.shape[0], WPAD - 2 * C), jnp.float32)
    o_ref[...] = jnp.concatenate([a, b, z], axis=1)

  return pl.pallas_call(
      body,
      out_shape=jax.ShapeDtypeStruct((N, WPAD), jnp.float32),
  )(p, b1_0, b1_1, W2_0, W2_1)


def _tc_fuse(p2, b2_0, b2_1, A1, a1b, A2, A3r):
  def body(p_ref, b0_ref, b1_ref, A1_ref, a1b_ref, A2_ref, A3_ref, o_ref):
    s = p_ref[0, :N] + p_ref[1, :N]
    o0 = s[:, :C] + b0_ref[...]
    o1 = s[:, C:2 * C] + b1_ref[...]

    def logsm(o):
      m = jnp.max(o, axis=1, keepdims=True)
      return o - m - jnp.log(jnp.sum(jnp.exp(o - m), axis=1, keepdims=True))

    z0 = logsm(o0)
    z1 = logsm(o1)

    def att(z):
      h = jnp.tanh(
          jnp.dot(z, A1_ref[...], preferred_element_type=jnp.float32)
          + a1b_ref[...])
      g = jnp.tanh(jnp.dot(h, A2_ref[...], preferred_element_type=jnp.float32))
      return jnp.sum(g * A3_ref[...], axis=1, keepdims=True)

    w0 = att(z0)
    w1 = att(z1)
    m = jnp.maximum(w0, w1)
    e0 = jnp.exp(w0 - m)
    e1 = jnp.exp(w1 - m)
    o_ref[...] = (e0 * z0 + e1 * z1) / (e0 + e1)

  return pl.pallas_call(
      body,
      out_shape=jax.ShapeDtypeStruct((N, C), jnp.float32),
  )(p2, b2_0, b2_1, A1, a1b, A2, A3r)


def kernel(x0, x1, edge_index, edge_weight, W1_0, b1_0, W2_0, b2_0,
           W1_1, b1_1, W2_1, b2_1, A1, a1b, A2, A3):
  src = edge_index[1].astype(jnp.int32)
  dst = edge_index[0].astype(jnp.int32)
  w = edge_weight.astype(jnp.float32)
  E = src.shape[0]
  n_chunks = _chunks_per_worker(E)
  pad = NW * n_chunks * CHUNK - E
  src2 = jnp.pad(src, (0, pad)).reshape(NW * n_chunks, CHUNK)
  dst2 = jnp.pad(dst, (0, pad)).reshape(NW * n_chunks, CHUNK)
  idx2 = jnp.stack([src2, dst2], axis=1).reshape(2 * NW * n_chunks, CHUNK)
  w2col = jnp.pad(w, (0, pad)).reshape(-1, 1)

  wexp = _tc_wexp(w2col)
  sup1 = _tc_support1(x0, x1, W1_0, W1_1)
  p1 = _spmm64(sup1, idx2, wexp)
  sup2 = _tc_support2(p1, b1_0.reshape(1, H), b1_1.reshape(1, H), W2_0, W2_1)
  p2 = _spmm32(sup2, idx2, wexp)
  return _tc_fuse(p2, b2_0.reshape(1, C), b2_1.reshape(1, C),
                  A1, a1b.reshape(1, FH), A2, A3.reshape(1, 2 * C))


# speedup vs baseline: 1.1602x; 1.1241x over previous
import functools

import jax
import jax.numpy as jnp
from jax import lax
from jax.experimental import pallas as pl
from jax.experimental.pallas import tpu as pltpu
from jax.experimental.pallas import tpu_sc as plsc

N = 10000
D = 128
H = 32
C = 16
FH = 64

NC = 2
NS = 16
NW = NC * NS
CHUNK = 128
LANES = 16
WPAD = 128

NP = 10240
ROWS_PER_TILE = NP // NS
GRP = 8


def _make_spmm(width: int, n_chunks: int):
  mesh = plsc.VectorSubcoreMesh(core_axis_name="c", subcore_axis_name="s")

  @functools.partial(
      pl.kernel,
      out_type=jax.ShapeDtypeStruct((NC, NP, WPAD), jnp.float32),
      mesh=mesh,
      scratch_types=[
          pltpu.VMEM_SHARED((NP, WPAD), jnp.float32),
          pltpu.VMEM((GRP, CHUNK), jnp.int32),
          pltpu.VMEM((GRP, CHUNK), jnp.int32),
          pltpu.VMEM((CHUNK, LANES), jnp.float32),
          pltpu.VMEM((CHUNK, WPAD), jnp.float32),
          pltpu.SemaphoreType.DMA,
      ],
  )
  def spmm(sup_hbm, src_hbm, dst_hbm, wexp_hbm, out_hbm,
           accum, src_v, dst_v, wexp_v, rows_v, sem):
    cid = lax.axis_index("c")
    sid = lax.axis_index("s")
    wid = cid * NS + sid
    row0 = sid * ROWS_PER_TILE

    zero16 = jnp.zeros((LANES,), jnp.float32)

    def zrow(i, carry):
      for jj in range(WPAD // LANES):
        rows_v[i, pl.ds(jj * LANES, LANES)] = zero16
      return carry

    lax.fori_loop(0, CHUNK, zrow, 0)
    for k in range(ROWS_PER_TILE // CHUNK):
      pltpu.sync_copy(rows_v, accum.at[pl.ds(row0 + k * CHUNK, CHUNK)])
    plsc.subcore_barrier()

    ebase = wid * n_chunks

    def group_body(g, carry):
      gbase = ebase + g * GRP
      pltpu.sync_copy(src_hbm.at[pl.ds(gbase, GRP)], src_v)
      pltpu.sync_copy(dst_hbm.at[pl.ds(gbase, GRP)], dst_v)

      def chunk_body(ch8, carry2):
        pltpu.async_copy(sup_hbm.at[src_v.at[ch8]], rows_v, sem).wait()
        pltpu.sync_copy(wexp_hbm.at[pl.ds((gbase + ch8) * CHUNK, CHUNK)],
                        wexp_v)
        for e in range(CHUNK):
          ws = wexp_v[e, :]
          for f in range(width // LANES):
            sl = pl.ds(f * LANES, LANES)
            rows_v[e, sl] = rows_v[e, sl] * ws
        pltpu.sync_copy(rows_v, accum.at[dst_v.at[ch8]], add=True)
        return carry2

      lax.fori_loop(0, GRP, chunk_body, 0)
      return carry

    lax.fori_loop(0, n_chunks // GRP, group_body, 0)
    plsc.subcore_barrier()

    for k in range(ROWS_PER_TILE // CHUNK):
      r = row0 + k * CHUNK
      pltpu.sync_copy(accum.at[pl.ds(r, CHUNK)], rows_v)
      pltpu.sync_copy(rows_v, out_hbm.at[cid, pl.ds(r, CHUNK)])

  return spmm


def _chunks_per_worker(E: int) -> int:
  return -(-(-(-E // (NW * CHUNK))) // 8) * 8


_N_CHUNKS = _chunks_per_worker(320000)
_spmm64 = _make_spmm(2 * H, _N_CHUNKS)
_spmm32 = _make_spmm(2 * C, _N_CHUNKS)


def _tc_support1(x0, x1, W1_0, W1_1):
  def body(x0_ref, x1_ref, w0_ref, w1_ref, o_ref):
    a = jnp.dot(x0_ref[...], w0_ref[...], preferred_element_type=jnp.float32)
    b = jnp.dot(x1_ref[...], w1_ref[...], preferred_element_type=jnp.float32)
    z = jnp.zeros((a.shape[0], WPAD - 2 * H), jnp.float32)
    o_ref[...] = jnp.concatenate([a, b, z], axis=1)

  return pl.pallas_call(
      body,
      out_shape=jax.ShapeDtypeStruct((N, WPAD), jnp.float32),
  )(x0, x1, W1_0, W1_1)


def _tc_wexp(w2col):
  BLK = 4096
  Ep = w2col.shape[0]

  def body(w_ref, o_ref):
    o_ref[...] = jnp.broadcast_to(w_ref[...], (BLK, LANES))

  return pl.pallas_call(
      body,
      grid=(Ep // BLK,),
      in_specs=[pl.BlockSpec((BLK, 1), lambda i: (i, 0))],
      out_specs=pl.BlockSpec((BLK, LANES), lambda i: (i, 0)),
      out_shape=jax.ShapeDtypeStruct((Ep, LANES), jnp.float32),
  )(w2col)


def _tc_support2(p, b1_0, b1_1, W2_0, W2_1):
  def body(p_ref, b0_ref, b1_ref, w0_ref, w1_ref, o_ref):
    s = p_ref[0, :N] + p_ref[1, :N]
    h0 = jnp.maximum(s[:, :H] + b0_ref[...], 0.0)
    h1 = jnp.maximum(s[:, H:2 * H] + b1_ref[...], 0.0)
    a = jnp.dot(h0, w0_ref[...], preferred_element_type=jnp.float32)
    b = jnp.dot(h1, w1_ref[...], preferred_element_type=jnp.float32)
    z = jnp.zeros((a.shape[0], WPAD - 2 * C), jnp.float32)
    o_ref[...] = jnp.concatenate([a, b, z], axis=1)

  return pl.pallas_call(
      body,
      out_shape=jax.ShapeDtypeStruct((N, WPAD), jnp.float32),
  )(p, b1_0, b1_1, W2_0, W2_1)


def _tc_fuse(p2, b2_0, b2_1, A1, a1b, A2, A3r):
  def body(p_ref, b0_ref, b1_ref, A1_ref, a1b_ref, A2_ref, A3_ref, o_ref):
    s = p_ref[0, :N] + p_ref[1, :N]
    o0 = s[:, :C] + b0_ref[...]
    o1 = s[:, C:2 * C] + b1_ref[...]

    def logsm(o):
      m = jnp.max(o, axis=1, keepdims=True)
      return o - m - jnp.log(jnp.sum(jnp.exp(o - m), axis=1, keepdims=True))

    z0 = logsm(o0)
    z1 = logsm(o1)

    def att(z):
      h = jnp.tanh(
          jnp.dot(z, A1_ref[...], preferred_element_type=jnp.float32)
          + a1b_ref[...])
      g = jnp.tanh(jnp.dot(h, A2_ref[...], preferred_element_type=jnp.float32))
      return jnp.sum(g * A3_ref[...], axis=1, keepdims=True)

    w0 = att(z0)
    w1 = att(z1)
    m = jnp.maximum(w0, w1)
    e0 = jnp.exp(w0 - m)
    e1 = jnp.exp(w1 - m)
    o_ref[...] = (e0 * z0 + e1 * z1) / (e0 + e1)

  return pl.pallas_call(
      body,
      out_shape=jax.ShapeDtypeStruct((N, C), jnp.float32),
  )(p2, b2_0, b2_1, A1, a1b, A2, A3r)


def kernel(x0, x1, edge_index, edge_weight, W1_0, b1_0, W2_0, b2_0,
           W1_1, b1_1, W2_1, b2_1, A1, a1b, A2, A3):
  src = edge_index[1].astype(jnp.int32)
  dst = edge_index[0].astype(jnp.int32)
  w = edge_weight.astype(jnp.float32)
  E = src.shape[0]
  n_chunks = _chunks_per_worker(E)
  pad = NW * n_chunks * CHUNK - E
  src2 = jnp.pad(src, (0, pad)).reshape(NW * n_chunks, CHUNK)
  dst2 = jnp.pad(dst, (0, pad)).reshape(NW * n_chunks, CHUNK)
  w2col = jnp.pad(w, (0, pad)).reshape(-1, 1)

  wexp = _tc_wexp(w2col)
  sup1 = _tc_support1(x0, x1, W1_0, W1_1)
  p1 = _spmm64(sup1, src2, dst2, wexp)
  sup2 = _tc_support2(p1, b1_0.reshape(1, H), b1_1.reshape(1, H), W2_0, W2_1)
  p2 = _spmm32(sup2, src2, dst2, wexp)
  return _tc_fuse(p2, b2_0.reshape(1, C), b2_1.reshape(1, C),
                  A1, a1b.reshape(1, FH), A2, A3.reshape(1, 2 * C))


# speedup vs baseline: 1.2798x; 1.1031x over previous
import functools

import jax
import jax.numpy as jnp
from jax import lax
from jax.experimental import pallas as pl
from jax.experimental.pallas import tpu as pltpu
from jax.experimental.pallas import tpu_sc as plsc

N = 10000
D = 128
H = 32
C = 16
FH = 64

NC = 2
NS = 16
NW = NC * NS
CHUNK = 128
LANES = 16
WPAD = 128

NP = 10240
ROWS_PER_TILE = NP // NS
GRP = 8


def _make_spmm(width: int, n_chunks: int):
  mesh = plsc.VectorSubcoreMesh(core_axis_name="c", subcore_axis_name="s")

  @functools.partial(
      pl.kernel,
      out_type=jax.ShapeDtypeStruct((NC, NP, WPAD), jnp.float32),
      mesh=mesh,
      scratch_types=[
          pltpu.VMEM_SHARED((NP, WPAD), jnp.float32),
          pltpu.VMEM((GRP, CHUNK), jnp.int32),
          pltpu.VMEM((GRP, CHUNK), jnp.int32),
          pltpu.VMEM((CHUNK, LANES), jnp.float32),
          pltpu.VMEM((CHUNK, WPAD), jnp.float32),
          pltpu.SemaphoreType.DMA,
          pltpu.SemaphoreType.DMA,
      ],
  )
  def spmm(sup_hbm, src_hbm, dst_hbm, wexp_hbm, out_hbm,
           accum, src_v, dst_v, wexp_v, rows_v, sem, sem_w):
    cid = lax.axis_index("c")
    sid = lax.axis_index("s")
    wid = cid * NS + sid
    row0 = sid * ROWS_PER_TILE

    zero16 = jnp.zeros((LANES,), jnp.float32)

    def zrow(i, carry):
      for jj in range(WPAD // LANES):
        rows_v[i, pl.ds(jj * LANES, LANES)] = zero16
      return carry

    lax.fori_loop(0, CHUNK, zrow, 0)
    for k in range(ROWS_PER_TILE // CHUNK):
      pltpu.sync_copy(rows_v, accum.at[pl.ds(row0 + k * CHUNK, CHUNK)])
    plsc.subcore_barrier()

    ebase = wid * n_chunks

    def group_body(g, carry):
      gbase = ebase + g * GRP
      pltpu.sync_copy(src_hbm.at[pl.ds(gbase, GRP)], src_v)
      pltpu.sync_copy(dst_hbm.at[pl.ds(gbase, GRP)], dst_v)

      def chunk_body(ch8, carry2):
        d = pltpu.async_copy(sup_hbm.at[src_v.at[ch8]], rows_v, sem)
        ew = pltpu.async_copy(
            wexp_hbm.at[pl.ds((gbase + ch8) * CHUNK, CHUNK)], wexp_v, sem_w)
        d.wait()
        ew.wait()
        for e in range(CHUNK):
          ws = wexp_v[e, :]
          for f in range(width // LANES):
            sl = pl.ds(f * LANES, LANES)
            rows_v[e, sl] = rows_v[e, sl] * ws
        pltpu.sync_copy(rows_v, accum.at[dst_v.at[ch8]], add=True)
        return carry2

      lax.fori_loop(0, GRP, chunk_body, 0)
      return carry

    lax.fori_loop(0, n_chunks // GRP, group_body, 0)
    plsc.subcore_barrier()

    for k in range(ROWS_PER_TILE // CHUNK):
      r = row0 + k * CHUNK
      pltpu.sync_copy(accum.at[pl.ds(r, CHUNK)], rows_v)
      pltpu.sync_copy(rows_v, out_hbm.at[cid, pl.ds(r, CHUNK)])

  return spmm


def _chunks_per_worker(E: int) -> int:
  return -(-(-(-E // (NW * CHUNK))) // 8) * 8


_N_CHUNKS = _chunks_per_worker(320000)
_spmm64 = _make_spmm(2 * H, _N_CHUNKS)
_spmm32 = _make_spmm(2 * C, _N_CHUNKS)


def _tc_support1(x0, x1, W1_0, W1_1):
  def body(x0_ref, x1_ref, w0_ref, w1_ref, o_ref):
    a = jnp.dot(x0_ref[...], w0_ref[...], preferred_element_type=jnp.float32)
    b = jnp.dot(x1_ref[...], w1_ref[...], preferred_element_type=jnp.float32)
    z = jnp.zeros((a.shape[0], WPAD - 2 * H), jnp.float32)
    o_ref[...] = jnp.concatenate([a, b, z], axis=1)

  return pl.pallas_call(
      body,
      out_shape=jax.ShapeDtypeStruct((N, WPAD), jnp.float32),
  )(x0, x1, W1_0, W1_1)


def _tc_wexp(w2col):
  BLK = 4096
  Ep = w2col.shape[0]

  def body(w_ref, o_ref):
    o_ref[...] = jnp.broadcast_to(w_ref[...], (BLK, LANES))

  return pl.pallas_call(
      body,
      grid=(Ep // BLK,),
      in_specs=[pl.BlockSpec((BLK, 1), lambda i: (i, 0))],
      out_specs=pl.BlockSpec((BLK, LANES), lambda i: (i, 0)),
      out_shape=jax.ShapeDtypeStruct((Ep, LANES), jnp.float32),
  )(w2col)


def _tc_support2(p, b1_0, b1_1, W2_0, W2_1):
  def body(p_ref, b0_ref, b1_ref, w0_ref, w1_ref, o_ref):
    s = p_ref[0, :N] + p_ref[1, :N]
    h0 = jnp.maximum(s[:, :H] + b0_ref[...], 0.0)
    h1 = jnp.maximum(s[:, H:2 * H] + b1_ref[...], 0.0)
    a = jnp.dot(h0, w0_ref[...], preferred_element_type=jnp.float32)
    b = jnp.dot(h1, w1_ref[...], preferred_element_type=jnp.float32)
    z = jnp.zeros((a.shape[0], WPAD - 2 * C), jnp.float32)
    o_ref[...] = jnp.concatenate([a, b, z], axis=1)

  return pl.pallas_call(
      body,
      out_shape=jax.ShapeDtypeStruct((N, WPAD), jnp.float32),
  )(p, b1_0, b1_1, W2_0, W2_1)


def _tc_fuse(p2, b2_0, b2_1, A1, a1b, A2, A3r):
  def body(p_ref, b0_ref, b1_ref, A1_ref, a1b_ref, A2_ref, A3_ref, o_ref):
    s = p_ref[0, :N] + p_ref[1, :N]
    o0 = s[:, :C] + b0_ref[...]
    o1 = s[:, C:2 * C] + b1_ref[...]

    def logsm(o):
      m = jnp.max(o, axis=1, keepdims=True)
      return o - m - jnp.log(jnp.sum(jnp.exp(o - m), axis=1, keepdims=True))

    z0 = logsm(o0)
    z1 = logsm(o1)

    def att(z):
      h = jnp.tanh(
          jnp.dot(z, A1_ref[...], preferred_element_type=jnp.float32)
          + a1b_ref[...])
      g = jnp.tanh(jnp.dot(h, A2_ref[...], preferred_element_type=jnp.float32))
      return jnp.sum(g * A3_ref[...], axis=1, keepdims=True)

    w0 = att(z0)
    w1 = att(z1)
    m = jnp.maximum(w0, w1)
    e0 = jnp.exp(w0 - m)
    e1 = jnp.exp(w1 - m)
    o_ref[...] = (e0 * z0 + e1 * z1) / (e0 + e1)

  return pl.pallas_call(
      body,
      out_shape=jax.ShapeDtypeStruct((N, C), jnp.float32),
  )(p2, b2_0, b2_1, A1, a1b, A2, A3r)


def kernel(x0, x1, edge_index, edge_weight, W1_0, b1_0, W2_0, b2_0,
           W1_1, b1_1, W2_1, b2_1, A1, a1b, A2, A3):
  src = edge_index[1].astype(jnp.int32)
  dst = edge_index[0].astype(jnp.int32)
  w = edge_weight.astype(jnp.float32)
  E = src.shape[0]
  n_chunks = _chunks_per_worker(E)
  pad = NW * n_chunks * CHUNK - E
  src2 = jnp.pad(src, (0, pad)).reshape(NW * n_chunks, CHUNK)
  dst2 = jnp.pad(dst, (0, pad)).reshape(NW * n_chunks, CHUNK)
  w2col = jnp.pad(w, (0, pad)).reshape(-1, 1)

  wexp = _tc_wexp(w2col)
  sup1 = _tc_support1(x0, x1, W1_0, W1_1)
  p1 = _spmm64(sup1, src2, dst2, wexp)
  sup2 = _tc_support2(p1, b1_0.reshape(1, H), b1_1.reshape(1, H), W2_0, W2_1)
  p2 = _spmm32(sup2, src2, dst2, wexp)
  return _tc_fuse(p2, b2_0.reshape(1, C), b2_1.reshape(1, C),
                  A1, a1b.reshape(1, FH), A2, A3.reshape(1, 2 * C))


# speedup vs baseline: 1.2891x; 1.0073x over previous
import functools

import jax
import jax.numpy as jnp
from jax import lax
from jax.experimental import pallas as pl
from jax.experimental.pallas import tpu as pltpu
from jax.experimental.pallas import tpu_sc as plsc

N = 10000
D = 128
H = 32
C = 16
FH = 64

NC = 2
NS = 16
NW = NC * NS
CHUNK = 128
LANES = 16
WPAD = 128

NP = 10240
ROWS_PER_TILE = NP // NS
GRP = 16


def _make_spmm(width: int, n_chunks: int):
  mesh = plsc.VectorSubcoreMesh(core_axis_name="c", subcore_axis_name="s")

  @functools.partial(
      pl.kernel,
      out_type=jax.ShapeDtypeStruct((NC, NP, WPAD), jnp.float32),
      mesh=mesh,
      scratch_types=[
          pltpu.VMEM_SHARED((NP, WPAD), jnp.float32),
          pltpu.VMEM((GRP, CHUNK), jnp.int32),
          pltpu.VMEM((GRP, CHUNK), jnp.int32),
          pltpu.VMEM((CHUNK, LANES), jnp.float32),
          pltpu.VMEM((CHUNK, WPAD), jnp.float32),
          pltpu.SemaphoreType.DMA,
          pltpu.SemaphoreType.DMA,
      ],
  )
  def spmm(sup_hbm, src_hbm, dst_hbm, wexp_hbm, out_hbm,
           accum, src_v, dst_v, wexp_v, rows_v, sem, sem_w):
    cid = lax.axis_index("c")
    sid = lax.axis_index("s")
    wid = cid * NS + sid
    row0 = sid * ROWS_PER_TILE

    zero16 = jnp.zeros((LANES,), jnp.float32)

    def zrow(i, carry):
      for jj in range(WPAD // LANES):
        rows_v[i, pl.ds(jj * LANES, LANES)] = zero16
      return carry

    lax.fori_loop(0, CHUNK, zrow, 0)
    for k in range(ROWS_PER_TILE // CHUNK):
      pltpu.sync_copy(rows_v, accum.at[pl.ds(row0 + k * CHUNK, CHUNK)])
    plsc.subcore_barrier()

    ebase = wid * n_chunks

    def group_body(g, carry):
      gbase = ebase + g * GRP
      pltpu.sync_copy(src_hbm.at[pl.ds(gbase, GRP)], src_v)
      pltpu.sync_copy(dst_hbm.at[pl.ds(gbase, GRP)], dst_v)

      def chunk_body(ch8, carry2):
        d = pltpu.async_copy(sup_hbm.at[src_v.at[ch8]], rows_v, sem)
        ew = pltpu.async_copy(
            wexp_hbm.at[pl.ds((gbase + ch8) * CHUNK, CHUNK)], wexp_v, sem_w)
        d.wait()
        ew.wait()
        for e in range(CHUNK):
          ws = wexp_v[e, :]
          for f in range(width // LANES):
            sl = pl.ds(f * LANES, LANES)
            rows_v[e, sl] = rows_v[e, sl] * ws
        pltpu.sync_copy(rows_v, accum.at[dst_v.at[ch8]], add=True)
        return carry2

      lax.fori_loop(0, GRP, chunk_body, 0)
      return carry

    lax.fori_loop(0, n_chunks // GRP, group_body, 0)
    plsc.subcore_barrier()

    for k in range(ROWS_PER_TILE // CHUNK):
      r = row0 + k * CHUNK
      pltpu.sync_copy(accum.at[pl.ds(r, CHUNK)], rows_v)
      pltpu.sync_copy(rows_v, out_hbm.at[cid, pl.ds(r, CHUNK)])

  return spmm


def _chunks_per_worker(E: int) -> int:
  return -(-(-(-E // (NW * CHUNK))) // 8) * 8


_N_CHUNKS = _chunks_per_worker(320000)
_spmm64 = _make_spmm(2 * H, _N_CHUNKS)
_spmm32 = _make_spmm(2 * C, _N_CHUNKS)


def _tc_support1(x0, x1, W1_0, W1_1):
  def body(x0_ref, x1_ref, w0_ref, w1_ref, o_ref):
    a = jnp.dot(x0_ref[...], w0_ref[...], preferred_element_type=jnp.float32)
    b = jnp.dot(x1_ref[...], w1_ref[...], preferred_element_type=jnp.float32)
    z = jnp.zeros((a.shape[0], WPAD - 2 * H), jnp.float32)
    o_ref[...] = jnp.concatenate([a, b, z], axis=1)

  return pl.pallas_call(
      body,
      out_shape=jax.ShapeDtypeStruct((N, WPAD), jnp.float32),
  )(x0, x1, W1_0, W1_1)


def _tc_wexp(w2col):
  BLK = 4096
  Ep = w2col.shape[0]

  def body(w_ref, o_ref):
    o_ref[...] = jnp.broadcast_to(w_ref[...], (BLK, LANES))

  return pl.pallas_call(
      body,
      grid=(Ep // BLK,),
      in_specs=[pl.BlockSpec((BLK, 1), lambda i: (i, 0))],
      out_specs=pl.BlockSpec((BLK, LANES), lambda i: (i, 0)),
      out_shape=jax.ShapeDtypeStruct((Ep, LANES), jnp.float32),
  )(w2col)


def _tc_support2(p, b1_0, b1_1, W2_0, W2_1):
  def body(p_ref, b0_ref, b1_ref, w0_ref, w1_ref, o_ref):
    s = p_ref[0, :N] + p_ref[1, :N]
    h0 = jnp.maximum(s[:, :H] + b0_ref[...], 0.0)
    h1 = jnp.maximum(s[:, H:2 * H] + b1_ref[...], 0.0)
    a = jnp.dot(h0, w0_ref[...], preferred_element_type=jnp.float32)
    b = jnp.dot(h1, w1_ref[...], preferred_element_type=jnp.float32)
    z = jnp.zeros((a.shape[0], WPAD - 2 * C), jnp.float32)
    o_ref[...] = jnp.concatenate([a, b, z], axis=1)

  return pl.pallas_call(
      body,
      out_shape=jax.ShapeDtypeStruct((N, WPAD), jnp.float32),
  )(p, b1_0, b1_1, W2_0, W2_1)


def _tc_fuse(p2, b2_0, b2_1, A1, a1b, A2, A3r):
  def body(p_ref, b0_ref, b1_ref, A1_ref, a1b_ref, A2_ref, A3_ref, o_ref):
    s = p_ref[0, :N] + p_ref[1, :N]
    o0 = s[:, :C] + b0_ref[...]
    o1 = s[:, C:2 * C] + b1_ref[...]

    def logsm(o):
      m = jnp.max(o, axis=1, keepdims=True)
      return o - m - jnp.log(jnp.sum(jnp.exp(o - m), axis=1, keepdims=True))

    z0 = logsm(o0)
    z1 = logsm(o1)

    def att(z):
      h = jnp.tanh(
          jnp.dot(z, A1_ref[...], preferred_element_type=jnp.float32)
          + a1b_ref[...])
      g = jnp.tanh(jnp.dot(h, A2_ref[...], preferred_element_type=jnp.float32))
      return jnp.sum(g * A3_ref[...], axis=1, keepdims=True)

    w0 = att(z0)
    w1 = att(z1)
    m = jnp.maximum(w0, w1)
    e0 = jnp.exp(w0 - m)
    e1 = jnp.exp(w1 - m)
    o_ref[...] = (e0 * z0 + e1 * z1) / (e0 + e1)

  return pl.pallas_call(
      body,
      out_shape=jax.ShapeDtypeStruct((N, C), jnp.float32),
  )(p2, b2_0, b2_1, A1, a1b, A2, A3r)


def kernel(x0, x1, edge_index, edge_weight, W1_0, b1_0, W2_0, b2_0,
           W1_1, b1_1, W2_1, b2_1, A1, a1b, A2, A3):
  src = edge_index[1].astype(jnp.int32)
  dst = edge_index[0].astype(jnp.int32)
  w = edge_weight.astype(jnp.float32)
  E = src.shape[0]
  n_chunks = _chunks_per_worker(E)
  pad = NW * n_chunks * CHUNK - E
  src2 = jnp.pad(src, (0, pad)).reshape(NW * n_chunks, CHUNK)
  dst2 = jnp.pad(dst, (0, pad)).reshape(NW * n_chunks, CHUNK)
  w2col = jnp.pad(w, (0, pad)).reshape(-1, 1)

  wexp = _tc_wexp(w2col)
  sup1 = _tc_support1(x0, x1, W1_0, W1_1)
  p1 = _spmm64(sup1, src2, dst2, wexp)
  sup2 = _tc_support2(p1, b1_0.reshape(1, H), b1_1.reshape(1, H), W2_0, W2_1)
  p2 = _spmm32(sup2, src2, dst2, wexp)
  return _tc_fuse(p2, b2_0.reshape(1, C), b2_1.reshape(1, C),
                  A1, a1b.reshape(1, FH), A2, A3.reshape(1, 2 * C))
